# baseline (speedup 1.0000x reference)
import functools

import jax
import jax.numpy as jnp
from jax import lax
from jax.experimental import pallas as pl
from jax.experimental.pallas import tpu as pltpu
from jax.experimental.pallas import tpu_sc as plsc

E = 8
TOPK = 2
DMODEL = 1024
DFF = 512
NGROUPS = 2
GROUP_SIZE = E // NGROUPS
N_TOKENS = 2048

_SC_INFO = plsc.get_sparse_core_info()
_NC = _SC_INFO.num_cores
_NS = _SC_INFO.num_subcores
_NW = _NC * _NS
_LANES = _SC_INFO.num_lanes
_TOK_PER_W = N_TOKENS // _NW


def _logits_kernel(x_ref, gw_ref, lt_ref, xbf_ref):
    xbf_ref[...] = x_ref[...].astype(jnp.bfloat16)
    lt_ref[...] = jax.lax.dot_general(
        gw_ref[...], x_ref[...], (((0,), (1,)), ((), ())),
        preferred_element_type=jnp.float32)


def _sc_route_kernel(lt_hbm, a0_hbm, a1_hbm, w0_hbm, w1_hbm,
                     lt_v, a0_v, a1_v, w0_v, w1_v):
    wid = lax.axis_index("s") * _NC + lax.axis_index("c")
    base = wid * _TOK_PER_W
    for e in range(E):
        pltpu.sync_copy(lt_hbm.at[e, pl.ds(base, _TOK_PER_W)], lt_v.at[e])
    for j in range(_TOK_PER_W // _LANES):
        sl = pl.ds(j * _LANES, _LANES)
        l = [lt_v[e, sl] for e in range(E)]
        m0 = l[0]
        a0 = jnp.full((_LANES,), 0, jnp.int32)
        for i in range(1, GROUP_SIZE):
            gt = l[i] > m0
            a0 = jnp.where(gt, i, a0)
            m0 = jnp.where(gt, l[i], m0)
        m1 = l[GROUP_SIZE]
        a1 = jnp.full((_LANES,), GROUP_SIZE, jnp.int32)
        for i in range(1, GROUP_SIZE):
            gt = l[GROUP_SIZE + i] > m1
            a1 = jnp.where(gt, GROUP_SIZE + i, a1)
            m1 = jnp.where(gt, l[GROUP_SIZE + i], m1)
        mm = jnp.maximum(m0, m1)
        e0 = jnp.exp(m0 - mm)
        e1 = jnp.exp(m1 - mm)
        s = e0 + e1
        a0_v[sl] = a0
        a1_v[sl] = a1
        w0_v[sl] = e0 / s
        w1_v[sl] = e1 / s
    pltpu.sync_copy(a0_v, a0_hbm.at[pl.ds(base, _TOK_PER_W)])
    pltpu.sync_copy(a1_v, a1_hbm.at[pl.ds(base, _TOK_PER_W)])
    pltpu.sync_copy(w0_v, w0_hbm.at[pl.ds(base, _TOK_PER_W)])
    pltpu.sync_copy(w1_v, w1_hbm.at[pl.ds(base, _TOK_PER_W)])


def _sc_route(lt):
    mesh = plsc.VectorSubcoreMesh(core_axis_name="c", subcore_axis_name="s")
    f = functools.partial(
        pl.kernel,
        mesh=mesh,
        out_type=(
            jax.ShapeDtypeStruct((N_TOKENS,), jnp.int32),
            jax.ShapeDtypeStruct((N_TOKENS,), jnp.int32),
            jax.ShapeDtypeStruct((N_TOKENS,), jnp.float32),
            jax.ShapeDtypeStruct((N_TOKENS,), jnp.float32),
        ),
        scratch_types=[
            pltpu.VMEM((E, _TOK_PER_W), jnp.float32),
            pltpu.VMEM((_TOK_PER_W,), jnp.int32),
            pltpu.VMEM((_TOK_PER_W,), jnp.int32),
            pltpu.VMEM((_TOK_PER_W,), jnp.float32),
            pltpu.VMEM((_TOK_PER_W,), jnp.float32),
        ],
    )(_sc_route_kernel)
    return f(lt)


def _ffn_kernel(a0_ref, a1_ref, w0_ref, w1_ref, x_ref, w1w_ref, w3w_ref,
                w2w_ref, out_ref):
    e = pl.program_id(0)
    comb = (jnp.where(a0_ref[...] == e, w0_ref[...], 0.0)
            + jnp.where(a1_ref[...] == e, w1_ref[...], 0.0))
    x = x_ref[...]
    a = jax.lax.dot_general(x, w1w_ref[0].astype(jnp.bfloat16),
                            (((1,), (1,)), ((), ())),
                            preferred_element_type=jnp.float32)
    b = jax.lax.dot_general(x, w3w_ref[0].astype(jnp.bfloat16),
                            (((1,), (1,)), ((), ())),
                            preferred_element_type=jnp.float32)
    h = (a * jax.nn.sigmoid(a) * b).astype(jnp.bfloat16)
    y = jax.lax.dot_general(h, w2w_ref[0].astype(jnp.bfloat16),
                            (((1,), (1,)), ((), ())),
                            preferred_element_type=jnp.float32)
    contrib = comb * y

    @pl.when(e == 0)
    def _():
        out_ref[...] = contrib

    @pl.when(e > 0)
    def _():
        out_ref[...] += contrib


@jax.jit
def kernel(hidden_states, gate_w, w1, w3, w2):
    lt, x_bf = pl.pallas_call(
        _logits_kernel,
        out_shape=(jax.ShapeDtypeStruct((E, N_TOKENS), jnp.float32),
                   jax.ShapeDtypeStruct((N_TOKENS, DMODEL), jnp.bfloat16)),
    )(hidden_states, gate_w)

    a0, a1, wt0, wt1 = _sc_route(lt)
    a0 = a0.reshape(N_TOKENS, 1)
    a1 = a1.reshape(N_TOKENS, 1)
    wt0 = wt0.reshape(N_TOKENS, 1)
    wt1 = wt1.reshape(N_TOKENS, 1)

    out = pl.pallas_call(
        _ffn_kernel,
        grid=(E,),
        in_specs=[
            pl.BlockSpec((N_TOKENS, 1), lambda e: (0, 0)),
            pl.BlockSpec((N_TOKENS, 1), lambda e: (0, 0)),
            pl.BlockSpec((N_TOKENS, 1), lambda e: (0, 0)),
            pl.BlockSpec((N_TOKENS, 1), lambda e: (0, 0)),
            pl.BlockSpec((N_TOKENS, DMODEL), lambda e: (0, 0)),
            pl.BlockSpec((1, DFF, DMODEL), lambda e: (e, 0, 0)),
            pl.BlockSpec((1, DFF, DMODEL), lambda e: (e, 0, 0)),
            pl.BlockSpec((1, DMODEL, DFF), lambda e: (e, 0, 0)),
        ],
        out_specs=pl.BlockSpec((N_TOKENS, DMODEL), lambda e: (0, 0)),
        out_shape=jax.ShapeDtypeStruct((N_TOKENS, DMODEL), jnp.float32),
    )(a0, a1, wt0, wt1, x_bf, w1, w3, w2)
    return out
